# baseline (speedup 1.0000x reference)
import jax
import jax.numpy as jnp
from jax import lax
from jax.experimental import pallas as pl
from jax.experimental.pallas import tpu as pltpu
from jax.experimental.pallas import tpu_sc as plsc

VOCAB = 1000000
N_EMBD = 64
B, L = 4096, 200

NW = 32
LB = 128
VOC2 = VOCAB // 2
NBUF = 2


def _gather_body(xt_hbm, tab_hbm, out_hbm, idx_v, g_v, rows_v, ot_v, gsems, osem):
    c = lax.axis_index("c")
    s = lax.axis_index("s")
    wid = s * 2 + c
    bbase = wid * LB

    pltpu.sync_copy(xt_hbm.at[:, pl.ds(bbase, LB)], idx_v)

    @pl.loop(0, L)
    def _shift(l):
        for cc in range(LB // 16):
            v = idx_v[l, pl.ds(cc * 16, 16)]
            g_v[l, pl.ds(cc * 16, 16)] = lax.shift_right_logical(v, 1)

    def fire(b, l):
        pltpu.async_copy(tab_hbm.at[g_v.at[l]], rows_v.at[b], gsems[b])

    def drain(b):
        pltpu.make_async_copy(
            tab_hbm.at[pl.ds(0, LB)], rows_v.at[b], gsems[b]
        ).wait()

    iota16 = lax.iota(jnp.int32, 16)

    def transpose_store(b, l):
        lv = jnp.full((16,), 0, jnp.int32) + l
        evs = [iota16 + (e0 * 16) for e0 in range(N_EMBD // 16)]

        @pl.loop(0, 16)
        def _d(d):
            jd = lax.bitwise_and(iota16 + d, 15)
            for j0 in range(LB // 16):
                jv = jd + (j0 * 16)
                pg = plsc.load_gather(idx_v, [lv, jv])
                pb = lax.shift_left(lax.bitwise_and(pg, 1), 6)
                for e0 in range(N_EMBD // 16):
                    cv = pb + evs[e0]
                    vals = plsc.load_gather(rows_v.at[b], [jv, cv])
                    plsc.store_scatter(ot_v, [evs[e0], jv], vals)

    for b in range(NBUF):
        fire(b, b)

    @pl.loop(0, (L - NBUF) // NBUF)
    def _t(t):
        for b in range(NBUF):
            l = t * NBUF + b
            drain(b)
            transpose_store(b, l)
            fire(b, l + NBUF)
            cp = pltpu.async_copy(
                ot_v, out_hbm.at[l, :, pl.ds(bbase, LB)], osem
            )
            cp.wait()

    for b in range(NBUF):
        l = L - NBUF + b
        drain(b)
        transpose_store(b, l)
        pltpu.async_copy(ot_v, out_hbm.at[l, :, pl.ds(bbase, LB)], osem).wait()


@jax.jit
def _embed_lookup(xt, tab_pairs):
    mesh = plsc.VectorSubcoreMesh(core_axis_name="c", subcore_axis_name="s")
    return pl.kernel(
        _gather_body,
        out_type=jax.ShapeDtypeStruct((L, N_EMBD, B), jnp.float32),
        mesh=mesh,
        scratch_types=[
            pltpu.VMEM((L, LB), jnp.int32),
            pltpu.VMEM((L, LB), jnp.int32),
            pltpu.VMEM((NBUF, LB, 128), jnp.float32),
            pltpu.VMEM((N_EMBD, LB), jnp.float32),
            [pltpu.SemaphoreType.DMA] * NBUF,
            pltpu.SemaphoreType.DMA,
        ],
        compiler_params=pltpu.CompilerParams(needs_layout_passes=False),
    )(xt, tab_pairs)


def kernel(x, token_embed):
    xt = x.astype(jnp.int32).T
    tab_pairs = jnp.concatenate(
        [token_embed[0::2], token_embed[1::2]], axis=1
    )
    out_t = _embed_lookup(xt, tab_pairs)
    return out_t.transpose(2, 0, 1)


# speedup vs baseline: 8.6849x; 8.6849x over previous
import jax
import jax.numpy as jnp
from jax import lax
from jax.experimental import pallas as pl
from jax.experimental.pallas import tpu as pltpu
from jax.experimental.pallas import tpu_sc as plsc

VOCAB = 1000000
N_EMBD = 64
B, L = 4096, 200

NW = 32
LB = 128
VOC2 = VOCAB // 2
NBUF = 2


def _gather_body(xt_hbm, tab_hbm, out_hbm, idx_v, g_v, rows_v, ot_v, gsems, osem):
    c = lax.axis_index("c")
    s = lax.axis_index("s")
    wid = s * 2 + c
    bbase = wid * LB

    pltpu.sync_copy(xt_hbm.at[:, pl.ds(bbase, LB)], idx_v)

    @pl.loop(0, L)
    def _shift(l):
        for cc in range(LB // 16):
            v = idx_v[l, pl.ds(cc * 16, 16)]
            g_v[l, pl.ds(cc * 16, 16)] = lax.shift_right_logical(v, 1)

    def fire(b, l):
        pltpu.async_copy(tab_hbm.at[g_v.at[l]], rows_v.at[b], gsems[b])

    def drain(b):
        pltpu.make_async_copy(
            tab_hbm.at[pl.ds(0, LB)], rows_v.at[b], gsems[b]
        ).wait()

    iota16 = lax.iota(jnp.int32, 16)

    def transpose_store(b, l):
        pcols = [
            lax.shift_left(
                lax.bitwise_and(idx_v[l, pl.ds(j0 * 16, 16)], 1), 6
            )
            for j0 in range(LB // 16)
        ]
        jvs = [iota16 + (j0 * 16) for j0 in range(LB // 16)]

        @pl.loop(0, 16)
        def _d(d):
            ed = lax.bitwise_and(iota16 + d, 15)
            for e0 in range(N_EMBD // 16):
                ev = ed + (e0 * 16)
                for j0 in range(LB // 16):
                    cv = pcols[j0] + ev
                    vals = plsc.load_gather(rows_v.at[b], [jvs[j0], cv])
                    plsc.store_scatter(ot_v, [ev, jvs[j0]], vals)

    for b in range(NBUF):
        fire(b, b)

    @pl.loop(0, (L - NBUF) // NBUF)
    def _t(t):
        for b in range(NBUF):
            l = t * NBUF + b
            drain(b)
            transpose_store(b, l)
            fire(b, l + NBUF)
            cp = pltpu.async_copy(
                ot_v, out_hbm.at[l, :, pl.ds(bbase, LB)], osem
            )
            cp.wait()

    for b in range(NBUF):
        l = L - NBUF + b
        drain(b)
        transpose_store(b, l)
        pltpu.async_copy(ot_v, out_hbm.at[l, :, pl.ds(bbase, LB)], osem).wait()


@jax.jit
def _embed_lookup(xt, tab_pairs):
    mesh = plsc.VectorSubcoreMesh(core_axis_name="c", subcore_axis_name="s")
    return pl.kernel(
        _gather_body,
        out_type=jax.ShapeDtypeStruct((L, N_EMBD, B), jnp.float32),
        mesh=mesh,
        scratch_types=[
            pltpu.VMEM((L, LB), jnp.int32),
            pltpu.VMEM((L, LB), jnp.int32),
            pltpu.VMEM((NBUF, LB, 128), jnp.float32),
            pltpu.VMEM((N_EMBD, LB), jnp.float32),
            [pltpu.SemaphoreType.DMA] * NBUF,
            pltpu.SemaphoreType.DMA,
        ],
        compiler_params=pltpu.CompilerParams(needs_layout_passes=False),
    )(xt, tab_pairs)


def kernel(x, token_embed):
    xt = x.astype(jnp.int32).T
    tab_pairs = token_embed.reshape(VOC2, 128)
    out_t = _embed_lookup(xt, tab_pairs)
    return out_t.transpose(2, 0, 1)
